# baseline (speedup 1.0000x reference)
import jax
import jax.numpy as jnp
from jax import lax
from jax.experimental import pallas as pl
from jax.experimental.pallas import tpu as pltpu
from jax.experimental.pallas import tpu_sc as plsc

DIM = 512
NW = 32
CHUNK = 48
NBUF = 5

SC_ROWS = 115200


def _gather_body(feats_hbm, table_hbm, out_hbm, idx_all,
                 buf0, buf1, buf2, buf3, buf4,
                 gs0, gs1, gs2, gs3, gs4,
                 ss0, ss1, ss2, ss3, ss4):
    wid = lax.axis_index("s") * 2 + lax.axis_index("c")
    n = feats_hbm.shape[0]
    per_w = n // NW
    chunks = per_w // CHUNK
    start = wid * per_w

    bufs = (buf0, buf1, buf2, buf3, buf4)
    gs = (gs0, gs1, gs2, gs3, gs4)
    ss = (ss0, ss1, ss2, ss3, ss4)

    def out_slc(i):
        return out_hbm.at[pl.ds(start + i * CHUNK, CHUNK)]

    def idx_slc(i):
        return idx_all.at[pl.ds(i * CHUNK, CHUNK)]

    def gather(i, b):
        pltpu.async_copy(table_hbm.at[idx_slc(i)], bufs[b], gs[b])

    def gather_wait(i, b):
        pltpu.make_async_copy(table_hbm.at[idx_slc(i)], bufs[b], gs[b]).wait()

    def store(i, b):
        pltpu.async_copy(bufs[b], out_slc(i), ss[b])

    def store_wait(i, b):
        pltpu.make_async_copy(bufs[b], out_slc(i), ss[b]).wait()

    pltpu.sync_copy(feats_hbm.at[pl.ds(start, per_w)], idx_all)

    gather(0, 0)
    gather(1, 1)
    gather(2, 2)
    gather_wait(0, 0)
    store(0, 0)
    gather(3, 3)
    gather_wait(1, 1)
    store(1, 1)
    gather(4, 4)
    gather_wait(2, 2)
    store(2, 2)
    store_wait(0, 0)
    gather(5, 0)

    def body(k, b, b2):
        gather_wait(k, b)
        store(k, b)
        store_wait(k - 2, b2)
        gather(k + 3, b2)

    def step(j, carry):
        k = 5 * j + 3
        body(k, 3, 1)
        body(k + 1, 4, 2)
        body(k + 2, 0, 3)
        body(k + 3, 1, 4)
        body(k + 4, 2, 0)
        return carry

    q = (chunks - 6) // 5
    lax.fori_loop(0, q, step, 0)
    for k in range(3 + 5 * q, chunks - 3):
        body(k, k % 5, (k - 2) % 5)

    for k in range(chunks - 3, chunks):
        gather_wait(k, k % 5)
        store(k, k % 5)
        if k - 2 >= 0:
            store_wait(k - 2, (k - 2) % 5)
    store_wait(chunks - 2, (chunks - 2) % 5)
    store_wait(chunks - 1, (chunks - 1) % 5)


def kernel(feats, table):
    B, T = feats.shape
    flat = feats.reshape(B * T)
    sc_flat = flat[:SC_ROWS]
    tc_flat = flat[SC_ROWS:]
    per_w = SC_ROWS // NW
    mesh = plsc.VectorSubcoreMesh(core_axis_name="c", subcore_axis_name="s")
    sc_out = pl.kernel(
        _gather_body,
        mesh=mesh,
        out_type=jax.ShapeDtypeStruct((SC_ROWS, DIM), jnp.float32),
        scratch_types=(
            [pltpu.VMEM((per_w,), jnp.int32)]
            + [pltpu.VMEM((CHUNK, DIM), jnp.float32)] * NBUF
            + [pltpu.SemaphoreType.DMA] * (2 * NBUF)
        ),
    )(sc_flat, table)
    tc_out = jnp.take(table, tc_flat, axis=0)
    return sc_out, tc_out


# speedup vs baseline: 1.0957x; 1.0957x over previous
import jax
import jax.numpy as jnp
from jax import lax
from jax.experimental import pallas as pl
from jax.experimental.pallas import tpu as pltpu
from jax.experimental.pallas import tpu_sc as plsc

DIM = 512
NW = 32
CHUNK = 48
NBUF = 5


def _gather_body(feats_hbm, table_hbm, out_hbm, idx_all,
                 buf0, buf1, buf2, buf3, buf4,
                 gs0, gs1, gs2, gs3, gs4,
                 ss0, ss1, ss2, ss3, ss4):
    wid = lax.axis_index("s") * 2 + lax.axis_index("c")
    n = feats_hbm.shape[0]
    per_w = n // NW
    chunks = per_w // CHUNK
    start = wid * per_w

    bufs = (buf0, buf1, buf2, buf3, buf4)
    gs = (gs0, gs1, gs2, gs3, gs4)
    ss = (ss0, ss1, ss2, ss3, ss4)

    def out_slc(i):
        return out_hbm.at[pl.ds(start + i * CHUNK, CHUNK)]

    def idx_slc(i):
        return idx_all.at[pl.ds(i * CHUNK, CHUNK)]

    def gather(i, b):
        pltpu.async_copy(table_hbm.at[idx_slc(i)], bufs[b], gs[b])

    def gather_wait(i, b):
        pltpu.make_async_copy(table_hbm.at[idx_slc(i)], bufs[b], gs[b]).wait()

    def store(i, b):
        pltpu.async_copy(bufs[b], out_slc(i), ss[b])

    def store_wait(i, b):
        pltpu.make_async_copy(bufs[b], out_slc(i), ss[b]).wait()

    pltpu.sync_copy(feats_hbm.at[pl.ds(start, per_w)], idx_all)

    gather(0, 0)
    gather(1, 1)
    gather(2, 2)

    gather_wait(0, 0)
    store(0, 0)
    gather(3, 3)
    gather_wait(1, 1)
    store(1, 1)
    gather(4, 4)

    def body(k, b, b2):
        store_wait(k - 2, b2)
        gather(k + 3, b2)
        gather_wait(k, b)
        store(k, b)

    body(2, 2, 0)

    def step(j, carry):
        k = 5 * j + 3
        body(k, 3, 1)
        body(k + 1, 4, 2)
        body(k + 2, 0, 3)
        body(k + 3, 1, 4)
        body(k + 4, 2, 0)
        return carry

    q = (chunks - 6) // 5
    lax.fori_loop(0, q, step, 0)
    for k in range(3 + 5 * q, chunks - 3):
        body(k, k % 5, (k - 2) % 5)

    for k in range(chunks - 3, chunks):
        gather_wait(k, k % 5)
        store(k, k % 5)
        store_wait(k - 2, (k - 2) % 5)
    store_wait(chunks - 2, (chunks - 2) % 5)
    store_wait(chunks - 1, (chunks - 1) % 5)


def kernel(feats, table):
    B, T = feats.shape
    flat = feats.reshape(B * T)
    per_w = (B * T) // NW
    mesh = plsc.VectorSubcoreMesh(core_axis_name="c", subcore_axis_name="s")
    out = pl.kernel(
        _gather_body,
        mesh=mesh,
        out_type=jax.ShapeDtypeStruct((B * T, DIM), jnp.float32),
        scratch_types=(
            [pltpu.VMEM((per_w,), jnp.int32)]
            + [pltpu.VMEM((CHUNK, DIM), jnp.float32)] * NBUF
            + [pltpu.SemaphoreType.DMA] * (2 * NBUF)
        ),
    )(flat, table)
    return out.reshape(B, T, DIM)
